# baseline (speedup 1.0000x reference)
import functools

import jax
import jax.numpy as jnp
from jax import lax
from jax.experimental import pallas as pl
from jax.experimental.pallas import tpu as pltpu
from jax.experimental.pallas import tpu_sc as plsc

_N = 10000
_E = 160000
_NT = 16
_NC = 2
_B = 128
_NB = 80
_NBT = _NT * _NB
_NPAD = 10240
_ZR = 128
_CW = 64
_FLUSH = 624
_RB = 1000


def _flush(acc, out, t, base):
    pltpu.sync_copy(acc.at[pl.ds(t * _FLUSH, _FLUSH)],
                    out.at[pl.ds(base + t * _FLUSH, _FLUSH)])

    @pl.when(t == _NT - 1)
    def _():
        pltpu.sync_copy(acc.at[pl.ds(_NT * _FLUSH, _N - _NT * _FLUSH)],
                        out.at[pl.ds(base + _NT * _FLUSH,
                                     _N - _NT * _FLUSH)])


def _deg_body(dstg, ones, zeros8, out, dstv, onesv, zbuf8, acc):
    c = lax.axis_index("c")
    t = lax.axis_index("s")
    pltpu.sync_copy(dstg.at[pl.ds(t * _NB, _NB)], dstv)
    pltpu.sync_copy(ones, onesv)
    pltpu.sync_copy(zeros8, zbuf8)
    pltpu.sync_copy(zbuf8, acc.at[pl.ds(t * (_NPAD // _NT), _NPAD // _NT)])
    plsc.subcore_barrier()
    half = _NB // _NC

    def batch(j, carry):
        pltpu.sync_copy(onesv, acc.at[dstv.at[c * half + j]], add=True)
        return carry

    lax.fori_loop(0, half, batch, 0)
    plsc.subcore_barrier()
    _flush(acc, out, t, c * _N)


def _deg_kernel(dstg, ones, zeros8):
    mesh = plsc.VectorSubcoreMesh(core_axis_name="c", subcore_axis_name="s")
    return pl.kernel(
        _deg_body,
        out_type=jax.ShapeDtypeStruct((_NC * _N, 8), jnp.float32),
        mesh=mesh,
        compiler_params=pltpu.CompilerParams(use_tc_tiling_on_sc=False),
        scratch_types=[
            pltpu.VMEM((_NB, _B), jnp.int32),
            pltpu.VMEM((_B, 8), jnp.float32),
            pltpu.VMEM((_NPAD // _NT, 8), jnp.float32),
            pltpu.VMEM_SHARED((_NPAD, 8), jnp.float32),
        ],
    )(dstg, ones, zeros8)


def _make_scatter_body(C):
    CPC = C // _NC
    NRING = 4

    def body(table, srcg, dstg, zeros, out, srcv, dstv, bufs, zbuf,
             acc, gsems, ssems):
        c = lax.axis_index("c")
        t = lax.axis_index("s")
        pltpu.sync_copy(dstg.at[pl.ds(t * _NB, _NB)], dstv)
        pltpu.sync_copy(srcg.at[pl.ds(t * _NB, _NB)], srcv)
        pltpu.sync_copy(zeros, zbuf)
        for k in range(CPC):
            g = c * CPC + k
            tbl = table.at[pl.ds(g * _N, _N)]
            for z in range(5):
                pltpu.sync_copy(zbuf,
                                acc.at[pl.ds(t * (_NPAD // _NT) + z * _ZR,
                                             _ZR)])
            plsc.subcore_barrier()

            def gath_start(j, r):
                pltpu.async_copy(tbl.at[srcv.at[j]], bufs.at[r], gsems[r])

            def gath_wait(j, r):
                pltpu.make_async_copy(tbl.at[srcv.at[j]], bufs.at[r],
                                      gsems[r]).wait()

            def scat_start(j, r):
                pltpu.async_copy(bufs.at[r], acc.at[dstv.at[j]], ssems[r],
                                 add=True)

            def scat_wait(j, r):
                pltpu.make_async_copy(bufs.at[r], acc.at[dstv.at[j]],
                                      ssems[r]).wait()

            gath_start(0, 0)
            gath_start(1, 1)

            def ring(ii, carry):
                for r4 in range(NRING):
                    jj = NRING * ii + r4
                    rg = (r4 + 2) % NRING

                    @pl.when(jj >= 2)
                    def _():
                        scat_wait(jj - 2, rg)

                    @pl.when(jj + 2 < _NB)
                    def _():
                        gath_start(jj + 2, rg)

                    gath_wait(jj, r4)
                    scat_start(jj, r4)
                return carry

            lax.fori_loop(0, _NB // NRING, ring, 0)
            scat_wait(_NB - 2, (_NB - 2) % NRING)
            scat_wait(_NB - 1, (_NB - 1) % NRING)
            plsc.subcore_barrier()
            _flush(acc, out, t, g * _N)
            plsc.subcore_barrier()

    return body


@functools.lru_cache(maxsize=None)
def _make_scatter_kernel(C):
    mesh = plsc.VectorSubcoreMesh(core_axis_name="c", subcore_axis_name="s")
    return pl.kernel(
        _make_scatter_body(C),
        out_type=jax.ShapeDtypeStruct((C * _N, _CW), jnp.float32),
        mesh=mesh,
        compiler_params=pltpu.CompilerParams(use_tc_tiling_on_sc=False),
        scratch_types=[
            pltpu.VMEM((_NB, _B), jnp.int32),
            pltpu.VMEM((_NB, _B), jnp.int32),
            pltpu.VMEM((4, _B, _CW), jnp.float32),
            pltpu.VMEM((_ZR, _CW), jnp.float32),
            pltpu.VMEM_SHARED((_NPAD, _CW), jnp.float32),
            [pltpu.SemaphoreType.DMA] * 4,
            [pltpu.SemaphoreType.DMA] * 4,
        ],
    )


def _scatter8(*args):
    return _make_scatter_kernel(8)(*args)


def _scatter4(*args):
    return _make_scatter_kernel(4)(*args)


def _dinv(dp_ref):
    return lax.rsqrt(dp_ref[:, 0:1] + dp_ref[:, 1:2] + 1.0)


def _k1_body(x_ref, w_ref, dp_ref, o_ref):
    h = jnp.dot(x_ref[...], w_ref[...], preferred_element_type=jnp.float32)
    h = h * _dinv(dp_ref)
    for cc in range(8):
        o_ref[cc] = h[:, cc * _CW:(cc + 1) * _CW]


def _mid_body(cin, cout, s_ref, h_ref, dp_ref, b_ref, w_ref, o_ref):
    dinv = _dinv(dp_ref)
    s = jnp.concatenate([s_ref[cc] for cc in range(cin)], axis=1)
    hp = jnp.concatenate([h_ref[cc] for cc in range(cin)], axis=1)
    xn = jnp.maximum(dinv * (s + hp) + b_ref[...], 0.0)
    h2 = jnp.dot(xn, w_ref[...], preferred_element_type=jnp.float32) * dinv
    for cc in range(cout):
        o_ref[cc] = h2[:, cc * _CW:(cc + 1) * _CW]


def _k4_body(s_ref, h_ref, dp_ref, b3_ref, x_ref, m1a_ref, m1b_ref,
             m1bias_ref, m2w_ref, m2b_ref, m3w_ref, m3b_ref, o_ref):
    dinv = _dinv(dp_ref)
    s = jnp.concatenate([s_ref[cc] for cc in range(4)], axis=1)
    hp = jnp.concatenate([h_ref[cc] for cc in range(4)], axis=1)
    emb = dinv * (s + hp) + b3_ref[...]
    z = jnp.dot(x_ref[...], m1a_ref[...], preferred_element_type=jnp.float32)
    z = z + jnp.dot(emb, m1b_ref[...], preferred_element_type=jnp.float32)
    z = jnp.maximum(z + m1bias_ref[...], 0.0)
    z = jnp.maximum(
        jnp.dot(z, m2w_ref[...], preferred_element_type=jnp.float32)
        + m2b_ref[...], 0.0)
    o_ref[...] = (jnp.dot(z, m3w_ref[...], preferred_element_type=jnp.float32)
                  + m3b_ref[...])


def _row_spec(shape3):
    return pl.BlockSpec(shape3, lambda i: (0, i, 0))


_DP_SPEC = pl.BlockSpec((_RB, 2), lambda i: (i, 0))


def _full_spec(shape):
    nd = len(shape)
    return pl.BlockSpec(shape, lambda i, _n=nd: (0,) * _n)


def _k1(x, w1, dp):
    return pl.pallas_call(
        _k1_body,
        grid=(_N // _RB,),
        in_specs=[
            pl.BlockSpec((_RB, 256), lambda i: (i, 0)),
            _full_spec((256, 512)),
            _DP_SPEC,
        ],
        out_specs=_row_spec((8, _RB, _CW)),
        out_shape=jax.ShapeDtypeStruct((8, _N, _CW), jnp.float32),
    )(x, w1, dp)


def _k_mid(cin, cout, s, h, dp, b, w):
    din, dout = cin * _CW, cout * _CW
    return pl.pallas_call(
        functools.partial(_mid_body, cin, cout),
        grid=(_N // _RB,),
        in_specs=[
            _row_spec((cin, _RB, _CW)),
            _row_spec((cin, _RB, _CW)),
            _DP_SPEC,
            _full_spec((1, din)),
            _full_spec((din, dout)),
        ],
        out_specs=_row_spec((cout, _RB, _CW)),
        out_shape=jax.ShapeDtypeStruct((cout, _N, _CW), jnp.float32),
    )(s, h, dp, b, w)


def _k4(s3, h3, dp, b3, x, m1a, m1b, m1bias, m2w, m2b, m3w, m3b):
    return pl.pallas_call(
        _k4_body,
        grid=(_N // _RB,),
        in_specs=[
            _row_spec((4, _RB, _CW)),
            _row_spec((4, _RB, _CW)),
            _DP_SPEC,
            _full_spec((1, 256)),
            pl.BlockSpec((_RB, 256), lambda i: (i, 0)),
            _full_spec((256, 512)),
            _full_spec((256, 512)),
            _full_spec((1, 512)),
            _full_spec((512, 256)),
            _full_spec((1, 256)),
            _full_spec((256, 1)),
            _full_spec((1, 1)),
        ],
        out_specs=pl.BlockSpec((_RB, 1), lambda i: (i, 0)),
        out_shape=jax.ShapeDtypeStruct((_N, 1), jnp.float32),
    )(s3, h3, dp, b3, x, m1a, m1b, m1bias, m2w, m2b, m3w, m3b)


def kernel(x, edge_indices, W1, b1, W2, b2, W3, b3,
           M1w, M1b, M2w, M2b, M3w, M3b):
    src = edge_indices[0].astype(jnp.int32)
    dst = edge_indices[1].astype(jnp.int32)
    src, dst = lax.sort_key_val(src, dst)

    padn = _NBT * _B - _E
    srcp = jnp.concatenate([src, jnp.zeros((padn,), jnp.int32)])
    dstp = jnp.concatenate([dst, jnp.full((padn,), _N, jnp.int32)])
    srcg = srcp.reshape(_NBT, _B)
    dstg = dstp.reshape(_NBT, _B)


    ones = jnp.zeros((_B, 8), jnp.float32).at[:, 0].set(1.0)
    zeros8 = jnp.zeros((_NPAD // _NT, 8), jnp.float32)
    zeros = jnp.zeros((_ZR, _CW), jnp.float32)

    degp = _deg_kernel(dstg, ones, zeros8)
    dp = jnp.stack([degp[:_N, 0], degp[_N:, 0]], axis=1)

    h1 = _k1(x, W1, dp)
    s1 = _scatter8(h1.reshape(8 * _N, _CW), srcg, dstg, zeros)
    h2 = _k_mid(8, 8, s1.reshape(8, _N, _CW), h1, dp, b1.reshape(1, -1), W2)
    s2 = _scatter8(h2.reshape(8 * _N, _CW), srcg, dstg, zeros)
    h3 = _k_mid(8, 4, s2.reshape(8, _N, _CW), h2, dp, b2.reshape(1, -1), W3)
    s3 = _scatter4(h3.reshape(4 * _N, _CW), srcg, dstg, zeros)
    out = _k4(s3.reshape(4, _N, _CW), h3, dp, b3.reshape(1, -1), x,
              M1w[:256], M1w[256:], M1b.reshape(1, -1),
              M2w, M2b.reshape(1, -1), M3w, M3b.reshape(1, 1))
    return out


# speedup vs baseline: 1.0702x; 1.0702x over previous
import functools

import numpy as np

import jax
import jax.numpy as jnp
from jax import lax
from jax.experimental import pallas as pl
from jax.experimental.pallas import tpu as pltpu
from jax.experimental.pallas import tpu_sc as plsc

_N = 10000
_E = 160000
_NT = 16
_NC = 2
_B = 128
_NB = 80
_NBT = _NT * _NB
_NPAD = 10240
_CW = 64
_FLUSH = 624
_RB = 2000
_STRIPE = _NPAD // _NT


def _interleave_perm(d):
    p = np.empty((d,), np.int32)
    for g in range(d // 32):
        for j in range(16):
            p[32 * g + 2 * j] = 32 * g + j
            p[32 * g + 2 * j + 1] = 32 * g + j + 16
    return p


_P512 = _interleave_perm(512)
_P256 = _interleave_perm(256)


def _flush(acc, out, t, base):
    pltpu.sync_copy(acc.at[pl.ds(t * _FLUSH, _FLUSH)],
                    out.at[pl.ds(base + t * _FLUSH, _FLUSH)])

    @pl.when(t == _NT - 1)
    def _():
        pltpu.sync_copy(acc.at[pl.ds(_NT * _FLUSH, _N - _NT * _FLUSH)],
                        out.at[pl.ds(base + _NT * _FLUSH,
                                     _N - _NT * _FLUSH)])


def _deg_body(dstg, ones, zeros8, out, dstv, onesv, zbuf8, acc):
    c = lax.axis_index("c")
    t = lax.axis_index("s")
    pltpu.sync_copy(dstg.at[pl.ds(t * _NB, _NB)], dstv)
    pltpu.sync_copy(ones, onesv)
    pltpu.sync_copy(zeros8, zbuf8)
    pltpu.sync_copy(zbuf8, acc.at[pl.ds(t * _STRIPE, _STRIPE)])
    plsc.subcore_barrier()
    half = _NB // _NC

    def batch(j, carry):
        pltpu.sync_copy(onesv, acc.at[dstv.at[c * half + j]], add=True)
        return carry

    lax.fori_loop(0, half, batch, 0)
    plsc.subcore_barrier()
    _flush(acc, out, t, c * _N)


def _deg_kernel(dstg, ones, zeros8):
    mesh = plsc.VectorSubcoreMesh(core_axis_name="c", subcore_axis_name="s")
    return pl.kernel(
        _deg_body,
        out_type=jax.ShapeDtypeStruct((_NC * _N, 8), jnp.float32),
        mesh=mesh,
        compiler_params=pltpu.CompilerParams(use_tc_tiling_on_sc=False),
        scratch_types=[
            pltpu.VMEM((_NB, _B), jnp.int32),
            pltpu.VMEM((_B, 8), jnp.float32),
            pltpu.VMEM((_STRIPE, 8), jnp.float32),
            pltpu.VMEM_SHARED((_NPAD, 8), jnp.float32),
        ],
    )(dstg, ones, zeros8)


def _make_scatter_body(C):
    CPC = C // _NC

    def body(table, srcg, dstg, out, srcv, dstv, gb, st, acc, gsems, ssems):
        c = lax.axis_index("c")
        t = lax.axis_index("s")
        pltpu.sync_copy(dstg.at[pl.ds(t * _NB, _NB)], dstv)
        pltpu.sync_copy(srcg.at[pl.ds(t * _NB, _NB)], srcv)
        himask = jnp.full((16,), -65536, jnp.int32)

        def expand(rg, rs, nrows):
            def row(i, carry):
                for h16 in range(2):
                    x = gb[rg, i, pl.ds(16 * h16, 16)]
                    lo = plsc.bitcast(x << 16, jnp.float32)
                    hi = plsc.bitcast(x & himask, jnp.float32)
                    st[rs, i, pl.ds(32 * h16, 16)] = lo
                    st[rs, i, pl.ds(32 * h16 + 16, 16)] = hi
                return carry

            lax.fori_loop(0, nrows, row, 0)

        for k in range(CPC):
            g = c * CPC + k
            tbl = table.at[pl.ds(g * _N, _N)]

            def init_rows(row0, nrows):
                pltpu.sync_copy(tbl.at[pl.ds(row0, nrows)],
                                gb.at[0, pl.ds(0, nrows)])
                expand(0, 0, nrows)
                pltpu.sync_copy(st.at[0, pl.ds(0, nrows)],
                                acc.at[pl.ds(row0, nrows)])

            @pl.when(t < _NT - 1)
            def _():
                for z in range(5):
                    init_rows(t * _STRIPE + z * _B, _B)

            @pl.when(t == _NT - 1)
            def _():
                for z in range(3):
                    init_rows((_NT - 1) * _STRIPE + z * _B, _B)
                init_rows((_NT - 1) * _STRIPE + 3 * _B, 16)
            plsc.subcore_barrier()

            def gath_start(j, r):
                pltpu.async_copy(tbl.at[srcv.at[j]], gb.at[r], gsems[r])

            def gath_wait(j, r):
                pltpu.make_async_copy(tbl.at[srcv.at[j]], gb.at[r],
                                      gsems[r]).wait()

            def scat_start(j, r):
                pltpu.async_copy(st.at[r], acc.at[dstv.at[j]], ssems[r],
                                 add=True)

            def scat_wait(j, r):
                pltpu.make_async_copy(st.at[r], acc.at[dstv.at[j]],
                                      ssems[r]).wait()

            gath_start(0, 0)
            gath_start(1, 1)
            gath_start(2, 2)

            def ring(ii, carry):
                for r4 in range(4):
                    jj = 4 * ii + r4
                    r2 = r4 % 2

                    @pl.when(jj + 3 < _NB)
                    def _():
                        gath_start(jj + 3, (r4 + 3) % 4)

                    @pl.when(jj >= 2)
                    def _():
                        scat_wait(jj - 2, r2)

                    gath_wait(jj, r4)
                    expand(r4, r2, _B)
                    scat_start(jj, r2)
                return carry

            lax.fori_loop(0, _NB // 4, ring, 0)
            scat_wait(_NB - 2, 0)
            scat_wait(_NB - 1, 1)
            plsc.subcore_barrier()
            _flush(acc, out, t, g * _N)
            plsc.subcore_barrier()

    return body


@functools.lru_cache(maxsize=None)
def _make_scatter_kernel(C):
    mesh = plsc.VectorSubcoreMesh(core_axis_name="c", subcore_axis_name="s")
    return pl.kernel(
        _make_scatter_body(C),
        out_type=jax.ShapeDtypeStruct((C * _N, _CW), jnp.float32),
        mesh=mesh,
        compiler_params=pltpu.CompilerParams(use_tc_tiling_on_sc=False,
                                             needs_layout_passes=False),
        scratch_types=[
            pltpu.VMEM((_NB, _B), jnp.int32),
            pltpu.VMEM((_NB, _B), jnp.int32),
            pltpu.VMEM((4, _B, _CW // 2), jnp.int32),
            pltpu.VMEM((2, _B, _CW), jnp.float32),
            pltpu.VMEM_SHARED((_NPAD, _CW), jnp.float32),
            [pltpu.SemaphoreType.DMA] * 4,
            [pltpu.SemaphoreType.DMA] * 2,
        ],
    )


def _scatter8(*args):
    return _make_scatter_kernel(8)(*args)


def _scatter4(*args):
    return _make_scatter_kernel(4)(*args)


def _dinv(dp_ref):
    return lax.rsqrt(dp_ref[:, 0:1] + dp_ref[:, 1:2] + 1.0)


def _k1_body(x_ref, w_ref, dp_ref, o_ref):
    h = jnp.dot(x_ref[...], w_ref[...], preferred_element_type=jnp.float32)
    h = h * _dinv(dp_ref)
    for cc in range(8):
        o_ref[cc] = h[:, cc * _CW:(cc + 1) * _CW].astype(jnp.bfloat16)


def _mid_body(cin, cout, s_ref, dp_ref, b_ref, w_ref, o_ref):
    dinv = _dinv(dp_ref)
    s = jnp.concatenate([s_ref[cc] for cc in range(cin)], axis=1)
    xn = jnp.maximum(dinv * s + b_ref[...], 0.0)
    h2 = jnp.dot(xn, w_ref[...], preferred_element_type=jnp.float32) * dinv
    for cc in range(cout):
        o_ref[cc] = h2[:, cc * _CW:(cc + 1) * _CW].astype(jnp.bfloat16)


def _k4_body(s_ref, dp_ref, b3_ref, x_ref, m1a_ref, m1b_ref,
             m1bias_ref, m2w_ref, m2b_ref, m3w_ref, m3b_ref, o_ref):
    dinv = _dinv(dp_ref)
    s = jnp.concatenate([s_ref[cc] for cc in range(4)], axis=1)
    emb = dinv * s + b3_ref[...]
    z = jnp.dot(x_ref[...], m1a_ref[...], preferred_element_type=jnp.float32)
    z = z + jnp.dot(emb, m1b_ref[...], preferred_element_type=jnp.float32)
    z = jnp.maximum(z + m1bias_ref[...], 0.0)
    z = jnp.maximum(
        jnp.dot(z, m2w_ref[...], preferred_element_type=jnp.float32)
        + m2b_ref[...], 0.0)
    o_ref[...] = (jnp.dot(z, m3w_ref[...], preferred_element_type=jnp.float32)
                  + m3b_ref[...])


def _row_spec(shape3):
    return pl.BlockSpec(shape3, lambda i: (0, i, 0))


_DP_SPEC = pl.BlockSpec((_RB, 2), lambda i: (i, 0))


def _full_spec(shape):
    nd = len(shape)
    return pl.BlockSpec(shape, lambda i, _n=nd: (0,) * _n)


def _k1(x, w1, dp):
    return pl.pallas_call(
        _k1_body,
        grid=(_N // _RB,),
        in_specs=[
            pl.BlockSpec((_RB, 256), lambda i: (i, 0)),
            _full_spec((256, 512)),
            _DP_SPEC,
        ],
        out_specs=_row_spec((8, _RB, _CW)),
        out_shape=jax.ShapeDtypeStruct((8, _N, _CW), jnp.bfloat16),
    )(x, w1, dp)


def _k_mid(cin, cout, s, dp, b, w):
    din, dout = cin * _CW, cout * _CW
    return pl.pallas_call(
        functools.partial(_mid_body, cin, cout),
        grid=(_N // _RB,),
        in_specs=[
            _row_spec((cin, _RB, _CW)),
            _DP_SPEC,
            _full_spec((1, din)),
            _full_spec((din, dout)),
        ],
        out_specs=_row_spec((cout, _RB, _CW)),
        out_shape=jax.ShapeDtypeStruct((cout, _N, _CW), jnp.bfloat16),
    )(s, dp, b, w)


def _k4(s3, dp, b3, x, m1a, m1b, m1bias, m2w, m2b, m3w, m3b):
    return pl.pallas_call(
        _k4_body,
        grid=(_N // _RB,),
        in_specs=[
            _row_spec((4, _RB, _CW)),
            _DP_SPEC,
            _full_spec((1, 256)),
            pl.BlockSpec((_RB, 256), lambda i: (i, 0)),
            _full_spec((256, 512)),
            _full_spec((256, 512)),
            _full_spec((1, 512)),
            _full_spec((512, 256)),
            _full_spec((1, 256)),
            _full_spec((256, 1)),
            _full_spec((1, 1)),
        ],
        out_specs=pl.BlockSpec((_RB, 1), lambda i: (i, 0)),
        out_shape=jax.ShapeDtypeStruct((_N, 1), jnp.float32),
    )(s3, dp, b3, x, m1a, m1b, m1bias, m2w, m2b, m3w, m3b)


def kernel(x, edge_indices, W1, b1, W2, b2, W3, b3,
           M1w, M1b, M2w, M2b, M3w, M3b):
    src = edge_indices[0].astype(jnp.int32)
    dst = edge_indices[1].astype(jnp.int32)

    padn = _NBT * _B - _E
    srcp = jnp.concatenate([src, jnp.zeros((padn,), jnp.int32)])
    dstp = jnp.concatenate([dst, jnp.full((padn,), _N, jnp.int32)])
    srcg = srcp.reshape(_NBT, _B)
    dstg = dstp.reshape(_NBT, _B)

    ones = jnp.zeros((_B, 8), jnp.float32).at[:, 0].set(1.0)
    zeros8 = jnp.zeros((_STRIPE, 8), jnp.float32)

    degp = _deg_kernel(dstg, ones, zeros8)
    dp = jnp.stack([degp[:_N, 0], degp[_N:, 0]], axis=1)

    W1p = W1[:, _P512]
    W2p = W2[:, _P512]
    W3p = W3[:, _P256]

    def pack(h, C):
        return jax.lax.bitcast_convert_type(
            h.reshape(C * _N, _CW // 2, 2), jnp.int32)

    h1 = _k1(x, W1p, dp)
    s1 = _scatter8(pack(h1, 8), srcg, dstg)
    h2 = _k_mid(8, 8, s1.reshape(8, _N, _CW), dp, b1.reshape(1, -1), W2p)
    s2 = _scatter8(pack(h2, 8), srcg, dstg)
    h3 = _k_mid(8, 4, s2.reshape(8, _N, _CW), dp, b2.reshape(1, -1), W3p)
    s3 = _scatter4(pack(h3, 4), srcg, dstg)
    out = _k4(s3.reshape(4, _N, _CW), dp, b3.reshape(1, -1), x,
              M1w[:256], M1w[256:], M1b.reshape(1, -1),
              M2w, M2b.reshape(1, -1), M3w, M3b.reshape(1, 1))
    return out


# speedup vs baseline: 1.3576x; 1.2685x over previous
import functools

import numpy as np

import jax
import jax.numpy as jnp
from jax import lax
from jax.experimental import pallas as pl
from jax.experimental.pallas import tpu as pltpu
from jax.experimental.pallas import tpu_sc as plsc

_N = 10000
_E = 160000
_NT = 16
_NC = 2
_B = 128
_NB = 80
_NBT = _NT * _NB
_NPAD = 10240
_CW = 64
_FLUSH = 624
_RB = 2000
_STRIPE = _NPAD // _NT


def _interleave_perm(d):
    p = np.empty((d,), np.int32)
    for g in range(d // 32):
        for j in range(16):
            p[32 * g + 2 * j] = 32 * g + j
            p[32 * g + 2 * j + 1] = 32 * g + j + 16
    return p


_P512 = _interleave_perm(512)
_P256 = _interleave_perm(256)


def _flush(acc, out, t, base):
    pltpu.sync_copy(acc.at[pl.ds(t * _FLUSH, _FLUSH)],
                    out.at[pl.ds(base + t * _FLUSH, _FLUSH)])

    @pl.when(t == _NT - 1)
    def _():
        pltpu.sync_copy(acc.at[pl.ds(_NT * _FLUSH, _N - _NT * _FLUSH)],
                        out.at[pl.ds(base + _NT * _FLUSH,
                                     _N - _NT * _FLUSH)])


def _deg_body(dstg, ones, zeros8, out, dstv, onesv, zbuf8, acc):
    c = lax.axis_index("c")
    t = lax.axis_index("s")
    pltpu.sync_copy(dstg.at[pl.ds(t * _NB, _NB)], dstv)
    pltpu.sync_copy(ones, onesv)
    pltpu.sync_copy(zeros8, zbuf8)
    pltpu.sync_copy(zbuf8, acc.at[pl.ds(t * _STRIPE, _STRIPE)])
    plsc.subcore_barrier()
    half = _NB // _NC

    def batch(j, carry):
        pltpu.sync_copy(onesv, acc.at[dstv.at[c * half + j]], add=True)
        return carry

    lax.fori_loop(0, half, batch, 0)
    plsc.subcore_barrier()
    _flush(acc, out, t, c * _N)


def _deg_kernel(dstg, ones, zeros8):
    mesh = plsc.VectorSubcoreMesh(core_axis_name="c", subcore_axis_name="s")
    return pl.kernel(
        _deg_body,
        out_type=jax.ShapeDtypeStruct((_NC * _N, 8), jnp.float32),
        mesh=mesh,
        compiler_params=pltpu.CompilerParams(use_tc_tiling_on_sc=False),
        scratch_types=[
            pltpu.VMEM((_NB, _B), jnp.int32),
            pltpu.VMEM((_B, 8), jnp.float32),
            pltpu.VMEM((_STRIPE, 8), jnp.float32),
            pltpu.VMEM_SHARED((_NPAD, 8), jnp.float32),
        ],
    )(dstg, ones, zeros8)


def _make_scatter_body(C):
    CPC = C // _NC

    def body(table, srcg, dstg, out, srcv, dstv, gb, st, acc, gsems, ssems):
        c = lax.axis_index("c")
        t = lax.axis_index("s")
        pltpu.sync_copy(dstg.at[pl.ds(t * _NB, _NB)], dstv)
        pltpu.sync_copy(srcg.at[pl.ds(t * _NB, _NB)], srcv)
        himask = jnp.full((16,), -65536, jnp.int32)

        def expand(rg, rs, nrows):
            @functools.partial(plsc.parallel_loop, 0, nrows, unroll=8)
            def row(i):
                for h16 in range(2):
                    x = gb[rg, i, pl.ds(16 * h16, 16)]
                    lo = plsc.bitcast(x << 16, jnp.float32)
                    hi = plsc.bitcast(x & himask, jnp.float32)
                    st[rs, i, pl.ds(32 * h16, 16)] = lo
                    st[rs, i, pl.ds(32 * h16 + 16, 16)] = hi

        for k in range(CPC):
            g = c * CPC + k
            tbl = table.at[pl.ds(g * _N, _N)]

            def init_rows(row0, nrows):
                pltpu.sync_copy(tbl.at[pl.ds(row0, nrows)],
                                gb.at[0, pl.ds(0, nrows)])
                expand(0, 0, nrows)
                pltpu.sync_copy(st.at[0, pl.ds(0, nrows)],
                                acc.at[pl.ds(row0, nrows)])

            @pl.when(t < _NT - 1)
            def _():
                for z in range(5):
                    init_rows(t * _STRIPE + z * _B, _B)

            @pl.when(t == _NT - 1)
            def _():
                for z in range(3):
                    init_rows((_NT - 1) * _STRIPE + z * _B, _B)
                init_rows((_NT - 1) * _STRIPE + 3 * _B, 16)
            plsc.subcore_barrier()

            def gath_start(j, r):
                pltpu.async_copy(tbl.at[srcv.at[j]], gb.at[r], gsems[r])

            def gath_wait(j, r):
                pltpu.make_async_copy(tbl.at[srcv.at[j]], gb.at[r],
                                      gsems[r]).wait()

            def scat_start(j, r):
                pltpu.async_copy(st.at[r], acc.at[dstv.at[j]], ssems[r],
                                 add=True)

            def scat_wait(j, r):
                pltpu.make_async_copy(st.at[r], acc.at[dstv.at[j]],
                                      ssems[r]).wait()

            gath_start(0, 0)
            gath_start(1, 1)
            gath_start(2, 2)

            def ring(ii, carry):
                for r4 in range(4):
                    jj = 4 * ii + r4
                    r2 = r4 % 2

                    @pl.when(jj + 3 < _NB)
                    def _():
                        gath_start(jj + 3, (r4 + 3) % 4)

                    @pl.when(jj >= 2)
                    def _():
                        scat_wait(jj - 2, r2)

                    gath_wait(jj, r4)
                    expand(r4, r2, _B)
                    scat_start(jj, r2)
                return carry

            lax.fori_loop(0, _NB // 4, ring, 0)
            scat_wait(_NB - 2, 0)
            scat_wait(_NB - 1, 1)
            plsc.subcore_barrier()
            _flush(acc, out, t, g * _N)
            plsc.subcore_barrier()

    return body


@functools.lru_cache(maxsize=None)
def _make_scatter_kernel(C):
    mesh = plsc.VectorSubcoreMesh(core_axis_name="c", subcore_axis_name="s")
    return pl.kernel(
        _make_scatter_body(C),
        out_type=jax.ShapeDtypeStruct((C * _N, _CW), jnp.float32),
        mesh=mesh,
        compiler_params=pltpu.CompilerParams(use_tc_tiling_on_sc=False,
                                             needs_layout_passes=False),
        scratch_types=[
            pltpu.VMEM((_NB, _B), jnp.int32),
            pltpu.VMEM((_NB, _B), jnp.int32),
            pltpu.VMEM((4, _B, _CW // 2), jnp.int32),
            pltpu.VMEM((2, _B, _CW), jnp.float32),
            pltpu.VMEM_SHARED((_NPAD, _CW), jnp.float32),
            [pltpu.SemaphoreType.DMA] * 4,
            [pltpu.SemaphoreType.DMA] * 2,
        ],
    )


def _scatter8(*args):
    return _make_scatter_kernel(8)(*args)


def _scatter4(*args):
    return _make_scatter_kernel(4)(*args)


def _dinv(dp_ref):
    return lax.rsqrt(dp_ref[:, 0:1] + dp_ref[:, 1:2] + 1.0)


def _k1_body(x_ref, w_ref, dp_ref, o_ref):
    h = jnp.dot(x_ref[...], w_ref[...], preferred_element_type=jnp.float32)
    h = h * _dinv(dp_ref)
    for cc in range(8):
        o_ref[cc] = h[:, cc * _CW:(cc + 1) * _CW].astype(jnp.bfloat16)


def _mid_body(cin, cout, s_ref, dp_ref, b_ref, w_ref, o_ref):
    dinv = _dinv(dp_ref)
    s = jnp.concatenate([s_ref[cc] for cc in range(cin)], axis=1)
    xn = jnp.maximum(dinv * s + b_ref[...], 0.0)
    h2 = jnp.dot(xn, w_ref[...], preferred_element_type=jnp.float32) * dinv
    for cc in range(cout):
        o_ref[cc] = h2[:, cc * _CW:(cc + 1) * _CW].astype(jnp.bfloat16)


def _k4_body(s_ref, dp_ref, b3_ref, x_ref, m1a_ref, m1b_ref,
             m1bias_ref, m2w_ref, m2b_ref, m3w_ref, m3b_ref, o_ref):
    dinv = _dinv(dp_ref)
    s = jnp.concatenate([s_ref[cc] for cc in range(4)], axis=1)
    emb = dinv * s + b3_ref[...]
    z = jnp.dot(x_ref[...], m1a_ref[...], preferred_element_type=jnp.float32)
    z = z + jnp.dot(emb, m1b_ref[...], preferred_element_type=jnp.float32)
    z = jnp.maximum(z + m1bias_ref[...], 0.0)
    z = jnp.maximum(
        jnp.dot(z, m2w_ref[...], preferred_element_type=jnp.float32)
        + m2b_ref[...], 0.0)
    o_ref[...] = (jnp.dot(z, m3w_ref[...], preferred_element_type=jnp.float32)
                  + m3b_ref[...])


def _row_spec(shape3):
    return pl.BlockSpec(shape3, lambda i: (0, i, 0))


_DP_SPEC = pl.BlockSpec((_RB, 2), lambda i: (i, 0))


def _full_spec(shape):
    nd = len(shape)
    return pl.BlockSpec(shape, lambda i, _n=nd: (0,) * _n)


def _k1(x, w1, dp):
    return pl.pallas_call(
        _k1_body,
        grid=(_N // _RB,),
        in_specs=[
            pl.BlockSpec((_RB, 256), lambda i: (i, 0)),
            _full_spec((256, 512)),
            _DP_SPEC,
        ],
        out_specs=_row_spec((8, _RB, _CW)),
        out_shape=jax.ShapeDtypeStruct((8, _N, _CW), jnp.bfloat16),
    )(x, w1, dp)


def _k_mid(cin, cout, s, dp, b, w):
    din, dout = cin * _CW, cout * _CW
    return pl.pallas_call(
        functools.partial(_mid_body, cin, cout),
        grid=(_N // _RB,),
        in_specs=[
            _row_spec((cin, _RB, _CW)),
            _DP_SPEC,
            _full_spec((1, din)),
            _full_spec((din, dout)),
        ],
        out_specs=_row_spec((cout, _RB, _CW)),
        out_shape=jax.ShapeDtypeStruct((cout, _N, _CW), jnp.bfloat16),
    )(s, dp, b, w)


def _k4(s3, dp, b3, x, m1a, m1b, m1bias, m2w, m2b, m3w, m3b):
    return pl.pallas_call(
        _k4_body,
        grid=(_N // _RB,),
        in_specs=[
            _row_spec((4, _RB, _CW)),
            _DP_SPEC,
            _full_spec((1, 256)),
            pl.BlockSpec((_RB, 256), lambda i: (i, 0)),
            _full_spec((256, 512)),
            _full_spec((256, 512)),
            _full_spec((1, 512)),
            _full_spec((512, 256)),
            _full_spec((1, 256)),
            _full_spec((256, 1)),
            _full_spec((1, 1)),
        ],
        out_specs=pl.BlockSpec((_RB, 1), lambda i: (i, 0)),
        out_shape=jax.ShapeDtypeStruct((_N, 1), jnp.float32),
    )(s3, dp, b3, x, m1a, m1b, m1bias, m2w, m2b, m3w, m3b)


def kernel(x, edge_indices, W1, b1, W2, b2, W3, b3,
           M1w, M1b, M2w, M2b, M3w, M3b):
    src = edge_indices[0].astype(jnp.int32)
    dst = edge_indices[1].astype(jnp.int32)

    padn = _NBT * _B - _E
    srcp = jnp.concatenate([src, jnp.zeros((padn,), jnp.int32)])
    dstp = jnp.concatenate([dst, jnp.full((padn,), _N, jnp.int32)])
    srcg = srcp.reshape(_NBT, _B)
    dstg = dstp.reshape(_NBT, _B)

    ones = jnp.zeros((_B, 8), jnp.float32).at[:, 0].set(1.0)
    zeros8 = jnp.zeros((_STRIPE, 8), jnp.float32)

    degp = _deg_kernel(dstg, ones, zeros8)
    dp = jnp.stack([degp[:_N, 0], degp[_N:, 0]], axis=1)

    W1p = W1[:, _P512]
    W2p = W2[:, _P512]
    W3p = W3[:, _P256]

    def pack(h, C):
        return jax.lax.bitcast_convert_type(
            h.reshape(C * _N, _CW // 2, 2), jnp.int32)

    h1 = _k1(x, W1p, dp)
    s1 = _scatter8(pack(h1, 8), srcg, dstg)
    h2 = _k_mid(8, 8, s1.reshape(8, _N, _CW), dp, b1.reshape(1, -1), W2p)
    s2 = _scatter8(pack(h2, 8), srcg, dstg)
    h3 = _k_mid(8, 4, s2.reshape(8, _N, _CW), dp, b2.reshape(1, -1), W3p)
    s3 = _scatter4(pack(h3, 4), srcg, dstg)
    out = _k4(s3.reshape(4, _N, _CW), dp, b3.reshape(1, -1), x,
              M1w[:256], M1w[256:], M1b.reshape(1, -1),
              M2w, M2b.reshape(1, -1), M3w, M3b.reshape(1, 1))
    return out
